# baseline (speedup 1.0000x reference)
import functools

import jax
import jax.numpy as jnp
from jax import lax
from jax.experimental import pallas as pl
from jax.experimental.pallas import tpu as pltpu
from jax.experimental.pallas import tpu_sc as plsc

N = 10000
NP = 10112
D = 128
E = 320000
NC = 2
NS = 16
NW = NC * NS
CHUNK = 112
EPW = 90
E_PAD = NW * EPW * CHUNK
ROWS_PER_TILE = NP // NS
BLK = 128


def _sc_body(want_deg, edges_hbm, table_hbm, zrows_hbm, zdeg_hbm,
             agg_out, deg_out, idx32, rows_v, ones_v, agg_sh, deg_sh,
             sem_a, sem_b):
  c = lax.axis_index("c")
  s = lax.axis_index("s")
  wid = s * NC + c
  tbase = s * ROWS_PER_TILE
  pltpu.sync_copy(zrows_hbm, agg_sh.at[pl.ds(tbase, ROWS_PER_TILE)])
  if want_deg:
    pltpu.sync_copy(zdeg_hbm, deg_sh.at[pl.ds(tbase, NP // NS)])
    for l in range(CHUNK // 16):
      ones_v[pl.ds(l * 16, 16)] = jnp.full((16,), 1.0, jnp.float32)
  pltpu.sync_copy(edges_hbm.at[wid], idx32)
  plsc.subcore_barrier()

  rows_a = rows_v.at[0]
  rows_b = rows_v.at[1]

  def gather(j, rv, sem):
    pltpu.async_copy(table_hbm.at[idx32.at[0, j]], rv, sem)

  def gwait(rv, sem):
    pltpu.make_async_copy(table_hbm.at[idx32.at[0, 0]], rv, sem).wait()

  def scatter(j, rv):
    pltpu.sync_copy(rv, agg_sh.at[idx32.at[1, j]], add=True)
    if want_deg:
      pltpu.sync_copy(ones_v, deg_sh.at[idx32.at[1, j]], add=True)

  gather(0, rows_a, sem_a)

  def pair(k, carry):
    j = 2 * k
    gwait(rows_a, sem_a)
    gather(j + 1, rows_b, sem_b)
    scatter(j, rows_a)
    gwait(rows_b, sem_b)
    gather(j + 2, rows_a, sem_a)
    scatter(j + 1, rows_b)
    return carry

  lax.fori_loop(0, EPW // 2 - 1, pair, 0)
  gwait(rows_a, sem_a)
  gather(EPW - 1, rows_b, sem_b)
  scatter(EPW - 2, rows_a)
  gwait(rows_b, sem_b)
  scatter(EPW - 1, rows_b)
  plsc.subcore_barrier()
  pltpu.sync_copy(agg_sh.at[pl.ds(tbase, ROWS_PER_TILE)],
                  agg_out.at[c, pl.ds(tbase, ROWS_PER_TILE)])
  if want_deg:
    pltpu.sync_copy(deg_sh.at[pl.ds(tbase, NP // NS)],
                    deg_out.at[c, pl.ds(tbase, NP // NS)])


@functools.lru_cache(maxsize=None)
def _make_sc(want_deg):
  out_type = [jax.ShapeDtypeStruct((NC, NP, D), jnp.float32),
              jax.ShapeDtypeStruct((NC, NP), jnp.float32)]
  mesh = plsc.VectorSubcoreMesh(core_axis_name="c", subcore_axis_name="s",
                                num_cores=NC, num_subcores=NS)
  scratch = [
      pltpu.VMEM((2, EPW, CHUNK), jnp.int32),
      pltpu.VMEM((2, CHUNK, D), jnp.float32),
      pltpu.VMEM((CHUNK,), jnp.float32),
      pltpu.VMEM_SHARED((NP, D), jnp.float32),
      pltpu.VMEM_SHARED((NP,), jnp.float32),
      pltpu.SemaphoreType.DMA,
      pltpu.SemaphoreType.DMA,
  ]
  return pl.kernel(functools.partial(_sc_body, want_deg), out_type=out_type,
                   mesh=mesh, scratch_types=scratch,
                   compiler_params=pltpu.CompilerParams(
                       use_tc_tiling_on_sc=False))


def _tc_layer12(a0, a1, dn, h, ctx, pt, wt, bt, out_ref):
  deg = jnp.sum(dn[...], axis=1, keepdims=True)
  rdeg = 1.0 / jnp.maximum(deg, 1.0)
  comb = (a0[...] + a1[...]) * rdeg + h[...]
  cx = ctx[...]
  c2 = jnp.sum(cx * cx, axis=1, keepdims=True)
  best = None
  sel = None
  for p in range(8):
    ptp = pt[p]
    cp = jnp.dot(cx, ptp, preferred_element_type=jnp.float32)
    p2 = jnp.sum(ptp * ptp, axis=0, keepdims=True)
    score = (c2 - 2.0 * cp) + p2
    if p == 0:
      best = score
      sel = jnp.zeros(score.shape, jnp.int32)
    else:
      m = score < best
      sel = jnp.where(m, p, sel)
      best = jnp.where(m, score, best)
  bias = bt[...]
  out = jnp.zeros(comb.shape, jnp.float32)
  for p in range(8):
    mm = jnp.dot(comb, wt[p], preferred_element_type=jnp.float32)
    out = out + jnp.where(sel == p, mm + bias[p:p + 1, :], 0.0)
  out_ref[...] = jnp.maximum(out, 0.0)


def _tc_layer3(a0, a1, dn, h, ctx, p2t, w2t, b2b, out_ref):
  deg = jnp.sum(dn[...], axis=1, keepdims=True)
  rdeg = 1.0 / jnp.maximum(deg, 1.0)
  comb = (a0[...] + a1[...]) * rdeg + h[...]
  cx = ctx[...]
  c2 = jnp.sum(cx * cx, axis=1, keepdims=True)
  pmat = p2t[...]
  cp = jnp.dot(cx, pmat, preferred_element_type=jnp.float32)
  p2 = jnp.sum(pmat * pmat, axis=0, keepdims=True)
  score = (c2 - 2.0 * cp) + p2
  best = score[:, 0:1]
  sel = jnp.zeros(best.shape, jnp.int32)
  for p in range(1, 8):
    sp = score[:, p:p + 1]
    m = sp < best
    sel = jnp.where(m, p, sel)
    best = jnp.where(m, sp, best)
  mm = jnp.dot(comb, w2t[...], preferred_element_type=jnp.float32) + b2b[0:1, :]
  idx8 = lax.broadcasted_iota(jnp.int32, mm.shape, 1)
  val = jnp.sum(jnp.where(idx8 == sel, mm, 0.0), axis=1, keepdims=True)
  sig = 1.0 / (1.0 + jnp.exp(-val))
  out_ref[...] = jnp.broadcast_to(sig, mm.shape)


def _node_spec(w):
  return pl.BlockSpec((BLK, w), lambda i: (i, 0))


def _full_spec(shape):
  nd = len(shape)
  return pl.BlockSpec(shape, lambda i, _nd=nd: (0,) * _nd)


_GRID = NP // BLK

_tc12_call = pl.pallas_call(
    _tc_layer12,
    grid=(_GRID,),
    in_specs=[_node_spec(D), _node_spec(D), _node_spec(NC),
              _node_spec(D), _node_spec(D),
              _full_spec((8, D, D)), _full_spec((8, D, D)), _full_spec((8, D))],
    out_specs=_node_spec(D),
    out_shape=jax.ShapeDtypeStruct((NP, D), jnp.float32),
)

_tc3_call = pl.pallas_call(
    _tc_layer3,
    grid=(_GRID,),
    in_specs=[_node_spec(D), _node_spec(D), _node_spec(NC),
              _node_spec(D), _node_spec(D),
              _full_spec((D, 8)), _full_spec((D, 8)), _full_spec((8, 8))],
    out_specs=_node_spec(8),
    out_shape=jax.ShapeDtypeStruct((NP, 8), jnp.float32),
)


@jax.jit
def kernel(x, edge_index, context, P0, W0, b0, P1, W1, b1, P2, W2, b2):
  src = edge_index[0]
  dst = edge_index[1]
  pad = E_PAD - E
  pad_idx = N + jnp.arange(pad, dtype=jnp.int32) % (NP - N)
  src3 = jnp.concatenate([src, pad_idx]).reshape(NW, EPW, CHUNK)
  dst3 = jnp.concatenate([dst, pad_idx]).reshape(NW, EPW, CHUNK)
  edges4 = jnp.stack([src3, dst3], axis=1)
  xp = jnp.pad(x, ((0, NP - N), (0, 0)))
  ctxp = jnp.pad(context, ((0, NP - N), (0, 0)))
  zrows = jnp.zeros((ROWS_PER_TILE, D), jnp.float32)
  zdeg = jnp.zeros((NP // NS,), jnp.float32)

  pt0 = jnp.transpose(P0, (1, 2, 0))
  wt0 = jnp.transpose(W0, (1, 2, 0))
  bt0 = b0.T
  pt1 = jnp.transpose(P1, (1, 2, 0))
  wt1 = jnp.transpose(W1, (1, 2, 0))
  bt1 = b1.T
  p2t = P2[0].T
  w2t = W2[0].T
  b2b = jnp.broadcast_to(b2.reshape(1, 8), (8, 8))

  agg1, degw = _make_sc(True)(edges4, xp, zrows, zdeg)
  dn = degw.T
  h1 = _tc12_call(agg1[0], agg1[1], dn, xp, ctxp, pt0, wt0, bt0)
  agg2, _ = _make_sc(False)(edges4, h1, zrows, zdeg)
  h2 = _tc12_call(agg2[0], agg2[1], dn, h1, ctxp, pt1, wt1, bt1)
  agg3, _ = _make_sc(False)(edges4, h2, zrows, zdeg)
  h3 = _tc3_call(agg3[0], agg3[1], dn, h2, ctxp, p2t, w2t, b2b)
  return (h1[:N], h2[:N], h3[:N, :1])


# speedup vs baseline: 1.0674x; 1.0674x over previous
import functools

import jax
import jax.numpy as jnp
from jax import lax
from jax.experimental import pallas as pl
from jax.experimental.pallas import tpu as pltpu
from jax.experimental.pallas import tpu_sc as plsc

N = 10000
NP = 10240
D = 128
E = 320000
NC = 2
NS = 16
NW = NC * NS
CHUNK = 96
EPW = 106
E_PAD = NW * EPW * CHUNK
ROWS_PER_TILE = NP // NS
BLK = 256


def _sc_body(want_deg, edges_hbm, table_hbm, zrows_hbm, zdeg_hbm,
             agg_out, deg_out, idx32, rows_v, ones_v, agg_sh, deg_sh,
             sem_a, sem_b):
  c = lax.axis_index("c")
  s = lax.axis_index("s")
  wid = s * NC + c
  tbase = s * ROWS_PER_TILE
  pltpu.sync_copy(zrows_hbm, agg_sh.at[pl.ds(tbase, ROWS_PER_TILE)])
  if want_deg:
    pltpu.sync_copy(zdeg_hbm, deg_sh.at[pl.ds(tbase, NP // NS)])
    for l in range(CHUNK // 16):
      ones_v[pl.ds(l * 16, 16)] = jnp.full((16,), 1.0, jnp.float32)
  pltpu.sync_copy(edges_hbm.at[wid], idx32)
  plsc.subcore_barrier()

  rows_a = rows_v.at[0]
  rows_b = rows_v.at[1]

  def gather(j, rv, sem):
    pltpu.async_copy(table_hbm.at[idx32.at[0, j]], rv, sem)

  def gwait(rv, sem):
    pltpu.make_async_copy(table_hbm.at[idx32.at[0, 0]], rv, sem).wait()

  def scatter(j, rv):
    pltpu.sync_copy(rv, agg_sh.at[idx32.at[1, j]], add=True)
    if want_deg:
      pltpu.sync_copy(ones_v, deg_sh.at[idx32.at[1, j]], add=True)

  gather(0, rows_a, sem_a)

  def pair(k, carry):
    j = 2 * k
    gwait(rows_a, sem_a)
    gather(j + 1, rows_b, sem_b)
    scatter(j, rows_a)
    gwait(rows_b, sem_b)
    gather(j + 2, rows_a, sem_a)
    scatter(j + 1, rows_b)
    return carry

  lax.fori_loop(0, EPW // 2 - 1, pair, 0)
  gwait(rows_a, sem_a)
  gather(EPW - 1, rows_b, sem_b)
  scatter(EPW - 2, rows_a)
  gwait(rows_b, sem_b)
  scatter(EPW - 1, rows_b)
  plsc.subcore_barrier()
  pltpu.sync_copy(agg_sh.at[pl.ds(tbase, ROWS_PER_TILE)],
                  agg_out.at[c, pl.ds(tbase, ROWS_PER_TILE)])
  if want_deg:
    pltpu.sync_copy(deg_sh.at[pl.ds(tbase, NP // NS)],
                    deg_out.at[c, pl.ds(tbase, NP // NS)])


@functools.lru_cache(maxsize=None)
def _make_sc(want_deg):
  out_type = [jax.ShapeDtypeStruct((NC, NP, D), jnp.float32),
              jax.ShapeDtypeStruct((NC, NP), jnp.float32)]
  mesh = plsc.VectorSubcoreMesh(core_axis_name="c", subcore_axis_name="s",
                                num_cores=NC, num_subcores=NS)
  scratch = [
      pltpu.VMEM((2, EPW, CHUNK), jnp.int32),
      pltpu.VMEM((2, CHUNK, D), jnp.float32),
      pltpu.VMEM((CHUNK,), jnp.float32),
      pltpu.VMEM_SHARED((NP, D), jnp.float32),
      pltpu.VMEM_SHARED((NP,), jnp.float32),
      pltpu.SemaphoreType.DMA,
      pltpu.SemaphoreType.DMA,
  ]
  return pl.kernel(functools.partial(_sc_body, want_deg), out_type=out_type,
                   mesh=mesh, scratch_types=scratch,
                   compiler_params=pltpu.CompilerParams(
                       use_tc_tiling_on_sc=False))


def _tc_layer12(a0, a1, dn, h, ctx, pt, wt, bt, out_ref):
  deg = jnp.sum(dn[...], axis=1, keepdims=True)
  rdeg = 1.0 / jnp.maximum(deg, 1.0)
  comb = (a0[...] + a1[...]) * rdeg + h[...]
  cx = ctx[...]
  c2 = jnp.sum(cx * cx, axis=1, keepdims=True)
  best = None
  sel = None
  for p in range(8):
    ptp = pt[p]
    cp = jnp.dot(cx, ptp, preferred_element_type=jnp.float32)
    p2 = jnp.sum(ptp * ptp, axis=0, keepdims=True)
    score = (c2 - 2.0 * cp) + p2
    if p == 0:
      best = score
      sel = jnp.zeros(score.shape, jnp.int32)
    else:
      m = score < best
      sel = jnp.where(m, p, sel)
      best = jnp.where(m, score, best)
  bias = bt[...]
  out = jnp.zeros(comb.shape, jnp.float32)
  for p in range(8):
    mm = jnp.dot(comb, wt[p], preferred_element_type=jnp.float32)
    out = out + jnp.where(sel == p, mm + bias[p:p + 1, :], 0.0)
  out_ref[...] = jnp.maximum(out, 0.0)


def _tc_layer3(a0, a1, dn, h, ctx, p2t, w2t, b2b, out_ref):
  deg = jnp.sum(dn[...], axis=1, keepdims=True)
  rdeg = 1.0 / jnp.maximum(deg, 1.0)
  comb = (a0[...] + a1[...]) * rdeg + h[...]
  cx = ctx[...]
  c2 = jnp.sum(cx * cx, axis=1, keepdims=True)
  pmat = p2t[...]
  cp = jnp.dot(cx, pmat, preferred_element_type=jnp.float32)
  p2 = jnp.sum(pmat * pmat, axis=0, keepdims=True)
  score = (c2 - 2.0 * cp) + p2
  best = score[:, 0:1]
  sel = jnp.zeros(best.shape, jnp.int32)
  for p in range(1, 8):
    sp = score[:, p:p + 1]
    m = sp < best
    sel = jnp.where(m, p, sel)
    best = jnp.where(m, sp, best)
  mm = jnp.dot(comb, w2t[...], preferred_element_type=jnp.float32) + b2b[0:1, :]
  idx8 = lax.broadcasted_iota(jnp.int32, mm.shape, 1)
  val = jnp.sum(jnp.where(idx8 == sel, mm, 0.0), axis=1, keepdims=True)
  sig = 1.0 / (1.0 + jnp.exp(-val))
  out_ref[...] = jnp.broadcast_to(sig, mm.shape)


def _node_spec(w):
  return pl.BlockSpec((BLK, w), lambda i: (i, 0))


def _full_spec(shape):
  nd = len(shape)
  return pl.BlockSpec(shape, lambda i, _nd=nd: (0,) * _nd)


_GRID = NP // BLK

_tc12_call = pl.pallas_call(
    _tc_layer12,
    grid=(_GRID,),
    in_specs=[_node_spec(D), _node_spec(D), _node_spec(NC),
              _node_spec(D), _node_spec(D),
              _full_spec((8, D, D)), _full_spec((8, D, D)), _full_spec((8, D))],
    out_specs=_node_spec(D),
    out_shape=jax.ShapeDtypeStruct((NP, D), jnp.float32),
)

_tc3_call = pl.pallas_call(
    _tc_layer3,
    grid=(_GRID,),
    in_specs=[_node_spec(D), _node_spec(D), _node_spec(NC),
              _node_spec(D), _node_spec(D),
              _full_spec((D, 8)), _full_spec((D, 8)), _full_spec((8, 8))],
    out_specs=_node_spec(8),
    out_shape=jax.ShapeDtypeStruct((NP, 8), jnp.float32),
)


@jax.jit
def kernel(x, edge_index, context, P0, W0, b0, P1, W1, b1, P2, W2, b2):
  src = edge_index[0]
  dst = edge_index[1]
  pad = E_PAD - E
  pad_idx = N + jnp.arange(pad, dtype=jnp.int32) % (NP - N)
  src3 = jnp.concatenate([src, pad_idx]).reshape(NW, EPW, CHUNK)
  dst3 = jnp.concatenate([dst, pad_idx]).reshape(NW, EPW, CHUNK)
  edges4 = jnp.stack([src3, dst3], axis=1)
  xp = jnp.pad(x, ((0, NP - N), (0, 0)))
  ctxp = jnp.pad(context, ((0, NP - N), (0, 0)))
  zrows = jnp.zeros((ROWS_PER_TILE, D), jnp.float32)
  zdeg = jnp.zeros((NP // NS,), jnp.float32)

  pt0 = jnp.transpose(P0, (1, 2, 0))
  wt0 = jnp.transpose(W0, (1, 2, 0))
  bt0 = b0.T
  pt1 = jnp.transpose(P1, (1, 2, 0))
  wt1 = jnp.transpose(W1, (1, 2, 0))
  bt1 = b1.T
  p2t = P2[0].T
  w2t = W2[0].T
  b2b = jnp.broadcast_to(b2.reshape(1, 8), (8, 8))

  agg1, degw = _make_sc(True)(edges4, xp, zrows, zdeg)
  dn = degw.T
  h1 = _tc12_call(agg1[0], agg1[1], dn, xp, ctxp, pt0, wt0, bt0)
  agg2, _ = _make_sc(False)(edges4, h1, zrows, zdeg)
  h2 = _tc12_call(agg2[0], agg2[1], dn, h1, ctxp, pt1, wt1, bt1)
  agg3, _ = _make_sc(False)(edges4, h2, zrows, zdeg)
  h3 = _tc3_call(agg3[0], agg3[1], dn, h2, ctxp, p2t, w2t, b2b)
  return (h1[:N], h2[:N], h3[:N, :1])
